# Initial kernel scaffold; baseline (speedup 1.0000x reference)
import jax
import jax.numpy as jnp
from jax.experimental import pallas as pl


def kernel(features, edge_index, W_in, b_in, W0, b0, W1, b1, W_out, b_out):
    raise NotImplementedError("write your pallas kernel here")


# speedup vs baseline: 3.6835x; 3.6835x over previous
import functools

import jax
import jax.numpy as jnp
from jax import lax
from jax.experimental import pallas as pl
from jax.experimental.pallas import tpu as pltpu
from jax.experimental.pallas import tpu_sc as plsc

N = 10000
D = 128
E = 320000
NP = 10240
ROWS = NP // 128
NC = 2
NS = 16
NW = NC * NS
K = 128
CH = 79
EW = K * CH
EP = EW * NW
IDXC = 1264
MB = 1280
GRID = NP // MB

_mesh = plsc.VectorSubcoreMesh(
    core_axis_name="c", subcore_axis_name="s", num_cores=NC, num_subcores=NS
)


def _zero_vmem_rows(ref, nrows):
    zero16 = jnp.zeros((16,), jnp.float32)

    def zrow(r, carry):
        for j in range(8):
            ref[r, pl.ds(j * 16, 16)] = zero16
        return carry

    lax.fori_loop(0, nrows, zrow, 0)


@functools.partial(
    pl.kernel,
    out_type=jax.ShapeDtypeStruct((NC, NP), jnp.float32),
    mesh=_mesh,
    scratch_types=[
        pltpu.VMEM((NP,), jnp.float32),
        pltpu.VMEM((IDXC,), jnp.int32),
        pltpu.VMEM((NP // NS,), jnp.float32),
        pltpu.VMEM((NP // NS,), jnp.float32),
        pltpu.VMEM_SHARED((NS, NP), jnp.float32),
    ],
    compiler_params=pltpu.CompilerParams(needs_layout_passes=False),
)
def _sc_deg(dst_hbm, out_hbm, hist, idxb, racc, rtmp, stage_sh):
    c = lax.axis_index("c")
    s = lax.axis_index("s")
    w = s * NC + c
    seg = NP // NS

    zero16 = jnp.zeros((16,), jnp.float32)

    def zel(i, carry):
        hist[pl.ds(pl.multiple_of(i * 16, 16), 16)] = zero16
        return carry

    lax.fori_loop(0, NP // 16, zel, 0)

    base = w * EW
    ones16 = jnp.ones((16,), jnp.float32)

    def outer(k, carry):
        off = pl.multiple_of(base + k * IDXC, 8)
        pltpu.sync_copy(dst_hbm.at[pl.ds(off, IDXC)], idxb)

        def inner(i, carry2):
            v = idxb[pl.ds(pl.multiple_of(i * 16, 16), 16)]
            plsc.addupdate_scatter(hist, [v], ones16)
            return carry2

        lax.fori_loop(0, IDXC // 16, inner, 0)
        return carry

    lax.fori_loop(0, EW // IDXC, outer, 0)

    pltpu.sync_copy(hist, stage_sh.at[s])
    plsc.subcore_barrier()

    cbase = pl.multiple_of(s * seg, 8)
    pltpu.sync_copy(stage_sh.at[0, pl.ds(cbase, seg)], racc)
    for k in range(1, NS):
        pltpu.sync_copy(stage_sh.at[k, pl.ds(cbase, seg)], rtmp)

        def addel(i, carry):
            sl = pl.ds(pl.multiple_of(i * 16, 16), 16)
            racc[sl] = racc[sl] + rtmp[sl]
            return carry

        lax.fori_loop(0, seg // 16, addel, 0)
    pltpu.sync_copy(racc, out_hbm.at[c, pl.ds(cbase, seg)])


@functools.partial(
    pl.kernel,
    out_type=jax.ShapeDtypeStruct((NC, NP, D), jnp.float32),
    mesh=_mesh,
    scratch_types=[
        pltpu.VMEM((K, D), jnp.float32),
        pltpu.VMEM((K,), jnp.int32),
        pltpu.VMEM((K,), jnp.int32),
        pltpu.VMEM_SHARED((NP, D), jnp.float32),
        pltpu.SemaphoreType.DMA,
    ],
    compiler_params=pltpu.CompilerParams(needs_layout_passes=False),
)
def _sc_seg(x_hbm, src_hbm, dst_hbm, out_hbm, rows, srcv, dstv, acc_sh, sem):
    c = lax.axis_index("c")
    s = lax.axis_index("s")
    w = s * NC + c

    _zero_vmem_rows(rows, K)
    srows = NP // NS
    for k in range(srows // K):
        off = pl.multiple_of(s * srows + k * K, 8)
        pltpu.sync_copy(rows, acc_sh.at[pl.ds(off, K)])
    plsc.subcore_barrier()

    base = w * EW

    def chunk(j, carry):
        off = pl.multiple_of(base + j * K, 8)
        pltpu.sync_copy(src_hbm.at[pl.ds(off, K)], srcv)
        pltpu.sync_copy(dst_hbm.at[pl.ds(off, K)], dstv)
        pltpu.async_copy(x_hbm.at[srcv], rows, sem).wait()
        pltpu.sync_copy(rows, acc_sh.at[dstv], add=True)
        return carry

    lax.fori_loop(0, CH, chunk, 0)
    plsc.subcore_barrier()

    for k in range(srows // K):
        off = pl.multiple_of(s * srows + k * K, 8)
        pltpu.sync_copy(acc_sh.at[pl.ds(off, K)], out_hbm.at[c, pl.ds(off, K)])


def _tc_in_body(x_ref, w_ref, b_ref, nc_ref, h_ref, xs_ref):
    h = jnp.dot(x_ref[...], w_ref[...], preferred_element_type=jnp.float32)
    h = h + b_ref[...]
    h_ref[...] = h
    xs_ref[...] = h * nc_ref[...]


def _tc_in(x, w, b, normc):
    return pl.pallas_call(
        _tc_in_body,
        grid=(GRID,),
        in_specs=[
            pl.BlockSpec((MB, D), lambda i: (i, 0)),
            pl.BlockSpec((D, D), lambda i: (0, 0)),
            pl.BlockSpec((1, D), lambda i: (0, 0)),
            pl.BlockSpec((MB, D), lambda i: (i, 0)),
        ],
        out_specs=[pl.BlockSpec((MB, D), lambda i: (i, 0))] * 2,
        out_shape=[jax.ShapeDtypeStruct((NP, D), jnp.float32)] * 2,
    )(x, w, b, normc)


def _tc_mid_body(p0_ref, p1_ref, nc_ref, w_ref, b_ref, h0_ref, x1_ref):
    y = (p0_ref[...] + p1_ref[...]) * nc_ref[...]
    t = jnp.dot(y, w_ref[...], preferred_element_type=jnp.float32) + b_ref[...]
    t = jnp.maximum(t, 0.0) + h0_ref[...]
    x1_ref[...] = t * nc_ref[...]


def _tc_mid(p0, p1, normc, w, b, h0):
    return pl.pallas_call(
        _tc_mid_body,
        grid=(GRID,),
        in_specs=[
            pl.BlockSpec((MB, D), lambda i: (i, 0)),
            pl.BlockSpec((MB, D), lambda i: (i, 0)),
            pl.BlockSpec((MB, D), lambda i: (i, 0)),
            pl.BlockSpec((D, D), lambda i: (0, 0)),
            pl.BlockSpec((1, D), lambda i: (0, 0)),
            pl.BlockSpec((MB, D), lambda i: (i, 0)),
        ],
        out_specs=pl.BlockSpec((MB, D), lambda i: (i, 0)),
        out_shape=jax.ShapeDtypeStruct((NP, D), jnp.float32),
    )(p0, p1, normc, w, b, h0)


def _tc_out_body(p0_ref, p1_ref, nc_ref, w1_ref, b1_ref, wo_ref, bo_ref, o_ref):
    y = (p0_ref[...] + p1_ref[...]) * nc_ref[...]
    h2 = jnp.dot(y, w1_ref[...], preferred_element_type=jnp.float32) + b1_ref[...]
    h2 = jnp.maximum(h2, 0.0)
    o_ref[...] = (
        jnp.dot(h2, wo_ref[...], preferred_element_type=jnp.float32) + bo_ref[...]
    )


def _tc_out(p0, p1, normc, w1, b1, wo, bo):
    return pl.pallas_call(
        _tc_out_body,
        grid=(GRID,),
        in_specs=[
            pl.BlockSpec((MB, D), lambda i: (i, 0)),
            pl.BlockSpec((MB, D), lambda i: (i, 0)),
            pl.BlockSpec((MB, D), lambda i: (i, 0)),
            pl.BlockSpec((D, D), lambda i: (0, 0)),
            pl.BlockSpec((1, D), lambda i: (0, 0)),
            pl.BlockSpec((D, D), lambda i: (0, 0)),
            pl.BlockSpec((1, D), lambda i: (0, 0)),
        ],
        out_specs=pl.BlockSpec((MB, D), lambda i: (i, 0)),
        out_shape=jax.ShapeDtypeStruct((NP, D), jnp.float32),
    )(p0, p1, normc, w1, b1, wo, bo)


def kernel(features, edge_index, W_in, b_in, W0, b0, W1, b1, W_out, b_out):
    src = edge_index[0].astype(jnp.int32)
    dst = edge_index[1].astype(jnp.int32)
    pad = EP - E
    srcp = jnp.concatenate([src, jnp.zeros((pad,), jnp.int32)])
    dstp = jnp.concatenate([dst, jnp.full((pad,), NP - 8, jnp.int32)])
    xp = jnp.pad(features, ((0, NP - N), (0, 0)))

    degp = _sc_deg(dstp)
    deg = degp[0] + degp[1]
    norm = lax.rsqrt(jnp.maximum(deg, 1.0))
    normc = jnp.broadcast_to(norm[:, None], (NP, D))

    h0, x0 = _tc_in(xp, W_in, b_in.reshape(1, D), normc)
    p = _sc_seg(x0, srcp, dstp)
    x1 = _tc_mid(p[0], p[1], normc, W0, b0.reshape(1, D), h0)
    p = _sc_seg(x1, srcp, dstp)
    out = _tc_out(p[0], p[1], normc, W1, b1.reshape(1, D), W_out, b_out.reshape(1, D))
    return out[:N]
